# baseline (speedup 1.0000x reference)
import functools

import jax
import jax.numpy as jnp
from jax import lax
from jax.experimental import pallas as pl
from jax.experimental.pallas import tpu as pltpu
from jax.experimental.pallas import tpu_sc as plsc

_NUM_CORES = 2
_NUM_SUBCORES = 16
_NUM_WORKERS = _NUM_CORES * _NUM_SUBCORES


@functools.lru_cache(maxsize=None)
def _make_sc_gather(total_rows: int, d_model: int):
    assert total_rows % (8 * _NUM_WORKERS) == 0
    rows_per_worker = total_rows // _NUM_WORKERS
    mesh = plsc.VectorSubcoreMesh(
        core_axis_name="c", subcore_axis_name="s",
        num_cores=_NUM_CORES, num_subcores=_NUM_SUBCORES)

    @functools.partial(
        pl.kernel,
        mesh=mesh,
        out_type=jax.ShapeDtypeStruct((total_rows, d_model), jnp.float32),
        scratch_types=[
            pltpu.VMEM((rows_per_worker,), jnp.int32),
            pltpu.VMEM((rows_per_worker, d_model), jnp.float32),
            pltpu.SemaphoreType.DMA,
        ],
    )
    def sc_gather(table_hbm, idx_hbm, out_hbm, idx_v, rows_v, sem):
        wid = lax.axis_index("s") * _NUM_CORES + lax.axis_index("c")
        base = wid * rows_per_worker
        pltpu.sync_copy(idx_hbm.at[pl.ds(base, rows_per_worker)], idx_v)
        pltpu.async_copy(table_hbm.at[idx_v], rows_v, sem).wait()
        pltpu.sync_copy(rows_v, out_hbm.at[pl.ds(base, rows_per_worker)])

    return sc_gather


def _matmul_body(lang_ref, x_ref, w_ref, b_ref, o_ref):
    del lang_ref
    acc = jax.lax.dot_general(
        x_ref[...].astype(jnp.bfloat16), w_ref[0].astype(jnp.bfloat16),
        dimension_numbers=(((1,), (1,)), ((), ())),
        preferred_element_type=jnp.float32)
    o_ref[...] = acc + b_ref[0]


@functools.lru_cache(maxsize=None)
def _make_tc_matmul(n_seq: int, seq_len: int, d_model: int, blk: int):
    n_tiles = seq_len // blk
    grid_spec = pltpu.PrefetchScalarGridSpec(
        num_scalar_prefetch=1,
        grid=(n_seq, n_tiles),
        in_specs=[
            pl.BlockSpec((blk, d_model),
                         lambda n, t, lang: (n * n_tiles + t, 0)),
            pl.BlockSpec((1, d_model, d_model),
                         lambda n, t, lang: (lang[n], 0, 0)),
            pl.BlockSpec((1, 1, d_model),
                         lambda n, t, lang: (lang[n], 0, 0)),
        ],
        out_specs=pl.BlockSpec((blk, d_model),
                               lambda n, t, lang: (n * n_tiles + t, 0)),
    )
    return pl.pallas_call(
        _matmul_body,
        grid_spec=grid_spec,
        out_shape=jax.ShapeDtypeStruct((n_seq * seq_len, d_model), jnp.float32),
    )


def kernel(sequences, embed_table, W, b):
    n_seq, seq_len = sequences.shape
    d_model = embed_table.shape[1]
    flat_idx = sequences.reshape(n_seq * seq_len).astype(jnp.int32)
    lang_ids = sequences[:, 0].astype(jnp.int32)
    b3 = b.reshape(b.shape[0], 1, d_model)

    gather = _make_sc_gather(seq_len, d_model)
    matmul = _make_tc_matmul(1, seq_len, d_model, 512)
    outs = []
    rows_chunks = [gather(embed_table, flat_idx[n * seq_len:(n + 1) * seq_len])
                   for n in range(n_seq)]
    for n in range(n_seq):
        outs.append(matmul(lang_ids[n:n + 1], rows_chunks[n], W, b3))
    return jnp.concatenate(outs, axis=0).reshape(n_seq, seq_len, d_model)


# speedup vs baseline: 1.1771x; 1.1771x over previous
import functools

import jax
import jax.numpy as jnp
from jax import lax
from jax.experimental import pallas as pl
from jax.experimental.pallas import tpu as pltpu
from jax.experimental.pallas import tpu_sc as plsc

_NUM_CORES = 2
_NUM_SUBCORES = 16
_NUM_WORKERS = _NUM_CORES * _NUM_SUBCORES


@functools.lru_cache(maxsize=None)
def _make_sc_gather(total_rows: int, d_model: int, n_chunks: int = 4):
    assert total_rows % (8 * _NUM_WORKERS) == 0
    rows_per_worker = total_rows // _NUM_WORKERS
    assert rows_per_worker % n_chunks == 0
    chunk = rows_per_worker // n_chunks
    mesh = plsc.VectorSubcoreMesh(
        core_axis_name="c", subcore_axis_name="s",
        num_cores=_NUM_CORES, num_subcores=_NUM_SUBCORES)

    @functools.partial(
        pl.kernel,
        mesh=mesh,
        out_type=jax.ShapeDtypeStruct((total_rows, d_model), jnp.float32),
        scratch_types=[
            pltpu.VMEM((rows_per_worker,), jnp.int32),
            pltpu.VMEM((2, chunk, d_model), jnp.float32),
            pltpu.SemaphoreType.DMA,
            pltpu.SemaphoreType.DMA,
        ],
    )
    def sc_gather(table_hbm, idx_hbm, out_hbm, idx_v, rows_v, gsem, wsem):
        wid = lax.axis_index("s") * _NUM_CORES + lax.axis_index("c")
        base = wid * rows_per_worker
        pltpu.sync_copy(idx_hbm.at[pl.ds(base, rows_per_worker)], idx_v)
        writebacks = [None, None]
        for c in range(n_chunks):
            buf = rows_v.at[c % 2]
            prev = writebacks[c % 2]
            if prev is not None:
                prev.wait()
            pltpu.async_copy(
                table_hbm.at[idx_v.at[pl.ds(c * chunk, chunk)]], buf,
                gsem).wait()
            wb = pltpu.make_async_copy(
                buf, out_hbm.at[pl.ds(base + c * chunk, chunk)], wsem)
            wb.start()
            writebacks[c % 2] = wb
        for wb in writebacks:
            if wb is not None:
                wb.wait()

    return sc_gather


def _matmul_body(lang_ref, x_ref, w_ref, b_ref, o_ref):
    del lang_ref
    acc = jax.lax.dot_general(
        x_ref[...].astype(jnp.bfloat16), w_ref[0].astype(jnp.bfloat16),
        dimension_numbers=(((1,), (1,)), ((), ())),
        preferred_element_type=jnp.float32)
    o_ref[...] = acc + b_ref[0]


@functools.lru_cache(maxsize=None)
def _make_tc_matmul(n_seq: int, seq_len: int, d_model: int, blk: int):
    n_tiles = seq_len // blk
    grid_spec = pltpu.PrefetchScalarGridSpec(
        num_scalar_prefetch=1,
        grid=(n_seq, n_tiles),
        in_specs=[
            pl.BlockSpec((blk, d_model),
                         lambda n, t, lang: (n * n_tiles + t, 0)),
            pl.BlockSpec((1, d_model, d_model),
                         lambda n, t, lang: (lang[n], 0, 0)),
            pl.BlockSpec((1, 1, d_model),
                         lambda n, t, lang: (lang[n], 0, 0)),
        ],
        out_specs=pl.BlockSpec((blk, d_model),
                               lambda n, t, lang: (n * n_tiles + t, 0)),
    )
    return pl.pallas_call(
        _matmul_body,
        grid_spec=grid_spec,
        out_shape=jax.ShapeDtypeStruct((n_seq * seq_len, d_model), jnp.float32),
    )


def kernel(sequences, embed_table, W, b):
    n_seq, seq_len = sequences.shape
    d_model = embed_table.shape[1]
    flat_idx = sequences.reshape(n_seq * seq_len).astype(jnp.int32)
    lang_ids = sequences[:, 0].astype(jnp.int32)
    rows = _make_sc_gather(n_seq * seq_len, d_model)(embed_table, flat_idx)
    out = _make_tc_matmul(n_seq, seq_len, d_model, 512)(
        lang_ids, rows, W, b.reshape(b.shape[0], 1, d_model))
    return out.reshape(n_seq, seq_len, d_model)


# speedup vs baseline: 1.2246x; 1.0403x over previous
import functools

import jax
import jax.numpy as jnp
from jax import lax
from jax.experimental import pallas as pl
from jax.experimental.pallas import tpu as pltpu
from jax.experimental.pallas import tpu_sc as plsc

_NUM_CORES = 2
_NUM_SUBCORES = 16
_NUM_WORKERS = _NUM_CORES * _NUM_SUBCORES


@functools.lru_cache(maxsize=None)
def _make_sc_gather(total_rows: int, d_model: int, n_chunks: int = 4):
    assert total_rows % (8 * _NUM_WORKERS) == 0
    rows_per_worker = total_rows // _NUM_WORKERS
    assert rows_per_worker % n_chunks == 0
    chunk = rows_per_worker // n_chunks
    mesh = plsc.VectorSubcoreMesh(
        core_axis_name="c", subcore_axis_name="s",
        num_cores=_NUM_CORES, num_subcores=_NUM_SUBCORES)

    @functools.partial(
        pl.kernel,
        mesh=mesh,
        out_type=jax.ShapeDtypeStruct((total_rows, d_model), jnp.float32),
        scratch_types=[
            pltpu.VMEM((rows_per_worker,), jnp.int32),
            pltpu.VMEM((rows_per_worker, d_model), jnp.float32),
            pltpu.SemaphoreType.DMA,
            pltpu.SemaphoreType.DMA,
        ],
    )
    def sc_gather(table_hbm, idx_hbm, out_hbm, idx_v, rows_v, gsem, wsem):
        wid = lax.axis_index("s") * _NUM_CORES + lax.axis_index("c")
        base = wid * rows_per_worker
        pltpu.sync_copy(idx_hbm.at[pl.ds(base, rows_per_worker)], idx_v)
        gathers, writebacks = [], []
        for c in range(n_chunks):
            g = pltpu.make_async_copy(
                table_hbm.at[idx_v.at[pl.ds(c * chunk, chunk)]],
                rows_v.at[pl.ds(c * chunk, chunk)], gsem)
            g.start()
            gathers.append(g)
        for c in range(n_chunks):
            gathers[c].wait()
            wb = pltpu.make_async_copy(
                rows_v.at[pl.ds(c * chunk, chunk)],
                out_hbm.at[pl.ds(base + c * chunk, chunk)], wsem)
            wb.start()
            writebacks.append(wb)
        for wb in writebacks:
            wb.wait()

    return sc_gather


def _matmul_body(lang_ref, x_ref, w_ref, b_ref, o_ref):
    del lang_ref
    acc = jax.lax.dot_general(
        x_ref[...].astype(jnp.bfloat16), w_ref[0].astype(jnp.bfloat16),
        dimension_numbers=(((1,), (1,)), ((), ())),
        preferred_element_type=jnp.float32)
    o_ref[...] = acc + b_ref[0]


@functools.lru_cache(maxsize=None)
def _make_tc_matmul(n_seq: int, seq_len: int, d_model: int, blk: int):
    n_tiles = seq_len // blk
    grid_spec = pltpu.PrefetchScalarGridSpec(
        num_scalar_prefetch=1,
        grid=(n_seq, n_tiles),
        in_specs=[
            pl.BlockSpec((blk, d_model),
                         lambda n, t, lang: (n * n_tiles + t, 0)),
            pl.BlockSpec((1, d_model, d_model),
                         lambda n, t, lang: (lang[n], 0, 0)),
            pl.BlockSpec((1, 1, d_model),
                         lambda n, t, lang: (lang[n], 0, 0)),
        ],
        out_specs=pl.BlockSpec((blk, d_model),
                               lambda n, t, lang: (n * n_tiles + t, 0)),
    )
    return pl.pallas_call(
        _matmul_body,
        grid_spec=grid_spec,
        out_shape=jax.ShapeDtypeStruct((n_seq * seq_len, d_model), jnp.float32),
    )


def kernel(sequences, embed_table, W, b):
    n_seq, seq_len = sequences.shape
    d_model = embed_table.shape[1]
    flat_idx = sequences.reshape(n_seq * seq_len).astype(jnp.int32)
    lang_ids = sequences[:, 0].astype(jnp.int32)
    rows = _make_sc_gather(n_seq * seq_len, d_model)(embed_table, flat_idx)
    out = _make_tc_matmul(n_seq, seq_len, d_model, 512)(
        lang_ids, rows, W, b.reshape(b.shape[0], 1, d_model))
    return out.reshape(n_seq, seq_len, d_model)


# speedup vs baseline: 1.2786x; 1.0441x over previous
import functools

import jax
import jax.numpy as jnp
from jax import lax
from jax.experimental import pallas as pl
from jax.experimental.pallas import tpu as pltpu
from jax.experimental.pallas import tpu_sc as plsc

_NUM_CORES = 2
_NUM_SUBCORES = 16
_NUM_WORKERS = _NUM_CORES * _NUM_SUBCORES


@functools.lru_cache(maxsize=None)
def _make_sc_gather(total_rows: int, d_model: int, n_chunks: int = 4):
    assert total_rows % (8 * _NUM_WORKERS) == 0
    rows_per_worker = total_rows // _NUM_WORKERS
    assert rows_per_worker % n_chunks == 0
    chunk = rows_per_worker // n_chunks
    mesh = plsc.VectorSubcoreMesh(
        core_axis_name="c", subcore_axis_name="s",
        num_cores=_NUM_CORES, num_subcores=_NUM_SUBCORES)

    @functools.partial(
        pl.kernel,
        mesh=mesh,
        out_type=jax.ShapeDtypeStruct((total_rows, d_model), jnp.float32),
        scratch_types=[
            pltpu.VMEM((rows_per_worker,), jnp.int32),
            pltpu.VMEM((rows_per_worker, d_model), jnp.float32),
            pltpu.SemaphoreType.DMA,
            pltpu.SemaphoreType.DMA,
        ],
    )
    def sc_gather(table_hbm, idx_hbm, out_hbm, idx_v, rows_v, gsem, wsem):
        wid = lax.axis_index("s") * _NUM_CORES + lax.axis_index("c")
        base = wid * rows_per_worker
        pltpu.sync_copy(idx_hbm.at[pl.ds(base, rows_per_worker)], idx_v)
        gathers, writebacks = [], []
        for c in range(n_chunks):
            g = pltpu.make_async_copy(
                table_hbm.at[idx_v.at[pl.ds(c * chunk, chunk)]],
                rows_v.at[pl.ds(c * chunk, chunk)], gsem)
            g.start()
            gathers.append(g)
        for c in range(n_chunks):
            gathers[c].wait()
            wb = pltpu.make_async_copy(
                rows_v.at[pl.ds(c * chunk, chunk)],
                out_hbm.at[pl.ds(base + c * chunk, chunk)], wsem)
            wb.start()
            writebacks.append(wb)
        for wb in writebacks:
            wb.wait()

    return sc_gather


def _matmul_body(lang_ref, x_ref, w_ref, b_ref, o_ref):
    del lang_ref
    acc = jax.lax.dot_general(
        x_ref[...], w_ref[0],
        dimension_numbers=(((1,), (1,)), ((), ())),
        preferred_element_type=jnp.float32)
    o_ref[...] = acc + b_ref[0]


@functools.lru_cache(maxsize=None)
def _make_tc_matmul(n_seq: int, seq_len: int, d_model: int, blk: int):
    n_tiles = seq_len // blk
    grid_spec = pltpu.PrefetchScalarGridSpec(
        num_scalar_prefetch=1,
        grid=(n_seq, n_tiles),
        in_specs=[
            pl.BlockSpec((blk, d_model),
                         lambda n, t, lang: (n * n_tiles + t, 0)),
            pl.BlockSpec((1, d_model, d_model),
                         lambda n, t, lang: (lang[n], 0, 0)),
            pl.BlockSpec((1, 1, d_model),
                         lambda n, t, lang: (lang[n], 0, 0)),
        ],
        out_specs=pl.BlockSpec((blk, d_model),
                               lambda n, t, lang: (n * n_tiles + t, 0)),
    )
    return pl.pallas_call(
        _matmul_body,
        grid_spec=grid_spec,
        out_shape=jax.ShapeDtypeStruct((n_seq * seq_len, d_model), jnp.float32),
    )


def kernel(sequences, embed_table, W, b):
    n_seq, seq_len = sequences.shape
    d_model = embed_table.shape[1]
    flat_idx = sequences.reshape(n_seq * seq_len).astype(jnp.int32)
    lang_ids = sequences[:, 0].astype(jnp.int32)
    rows = _make_sc_gather(n_seq * seq_len, d_model)(embed_table, flat_idx)
    out = _make_tc_matmul(n_seq, seq_len, d_model, 1024)(
        lang_ids, rows, W, b.reshape(b.shape[0], 1, d_model))
    return out.reshape(n_seq, seq_len, d_model)


# speedup vs baseline: 1.3364x; 1.0452x over previous
import functools

import jax
import jax.numpy as jnp
from jax import lax
from jax.experimental import pallas as pl
from jax.experimental.pallas import tpu as pltpu
from jax.experimental.pallas import tpu_sc as plsc

_NUM_CORES = 2
_NUM_SUBCORES = 16
_NUM_WORKERS = _NUM_CORES * _NUM_SUBCORES


@functools.lru_cache(maxsize=None)
def _make_sc_gather(total_rows: int, d_model: int, n_chunks: int = 4):
    assert total_rows % (8 * _NUM_WORKERS) == 0
    rows_per_worker = total_rows // _NUM_WORKERS
    assert rows_per_worker % n_chunks == 0
    chunk = rows_per_worker // n_chunks
    mesh = plsc.VectorSubcoreMesh(
        core_axis_name="c", subcore_axis_name="s",
        num_cores=_NUM_CORES, num_subcores=_NUM_SUBCORES)

    @functools.partial(
        pl.kernel,
        mesh=mesh,
        out_type=jax.ShapeDtypeStruct((total_rows, d_model), jnp.float32),
        scratch_types=[
            pltpu.VMEM((rows_per_worker,), jnp.int32),
            pltpu.VMEM((rows_per_worker, d_model), jnp.float32),
            pltpu.SemaphoreType.DMA,
            pltpu.SemaphoreType.DMA,
        ],
    )
    def sc_gather(table_hbm, idx_hbm, out_hbm, idx_v, rows_v, gsem, wsem):
        wid = lax.axis_index("s") * _NUM_CORES + lax.axis_index("c")
        base = wid * rows_per_worker
        pltpu.sync_copy(idx_hbm.at[pl.ds(base, rows_per_worker)], idx_v)
        gathers, writebacks = [], []
        for c in range(n_chunks):
            g = pltpu.make_async_copy(
                table_hbm.at[idx_v.at[pl.ds(c * chunk, chunk)]],
                rows_v.at[pl.ds(c * chunk, chunk)], gsem)
            g.start()
            gathers.append(g)
        for c in range(n_chunks):
            gathers[c].wait()
            wb = pltpu.make_async_copy(
                rows_v.at[pl.ds(c * chunk, chunk)],
                out_hbm.at[pl.ds(base + c * chunk, chunk)], wsem)
            wb.start()
            writebacks.append(wb)
        for wb in writebacks:
            wb.wait()

    return sc_gather


def _matmul_body(lang_ref, x_ref, w_ref, b_ref, o_ref):
    del lang_ref
    acc = jax.lax.dot_general(
        x_ref[...], w_ref[0],
        dimension_numbers=(((1,), (1,)), ((), ())),
        preferred_element_type=jnp.float32)
    o_ref[...] = acc + b_ref[0]


@functools.lru_cache(maxsize=None)
def _make_tc_matmul(n_seq: int, seq_len: int, d_model: int, blk: int):
    n_tiles = seq_len // blk
    grid_spec = pltpu.PrefetchScalarGridSpec(
        num_scalar_prefetch=1,
        grid=(n_seq, n_tiles),
        in_specs=[
            pl.BlockSpec((blk, d_model),
                         lambda n, t, lang: (n * n_tiles + t, 0)),
            pl.BlockSpec((1, d_model, d_model),
                         lambda n, t, lang: (lang[n], 0, 0)),
            pl.BlockSpec((1, 1, d_model),
                         lambda n, t, lang: (lang[n], 0, 0)),
        ],
        out_specs=pl.BlockSpec((blk, d_model),
                               lambda n, t, lang: (n * n_tiles + t, 0)),
    )
    return pl.pallas_call(
        _matmul_body,
        grid_spec=grid_spec,
        out_shape=jax.ShapeDtypeStruct((n_seq * seq_len, d_model), jnp.float32),
    )


def kernel(sequences, embed_table, W, b):
    n_seq, seq_len = sequences.shape
    d_model = embed_table.shape[1]
    flat_idx = sequences.reshape(n_seq * seq_len).astype(jnp.int32)
    lang_ids = sequences[:, 0].astype(jnp.int32)
    rows = _make_sc_gather(n_seq * seq_len, d_model)(embed_table, flat_idx)
    out = _make_tc_matmul(n_seq, seq_len, d_model, 2048)(
        lang_ids, rows, W, b.reshape(b.shape[0], 1, d_model))
    return out.reshape(n_seq, seq_len, d_model)


# speedup vs baseline: 1.3374x; 1.0008x over previous
import functools

import jax
import jax.numpy as jnp
from jax import lax
from jax.experimental import pallas as pl
from jax.experimental.pallas import tpu as pltpu
from jax.experimental.pallas import tpu_sc as plsc

_NUM_CORES = 2
_NUM_SUBCORES = 16
_NUM_WORKERS = _NUM_CORES * _NUM_SUBCORES


@functools.lru_cache(maxsize=None)
def _make_sc_gather(total_rows: int, d_model: int, n_chunks: int = 4):
    assert total_rows % (8 * _NUM_WORKERS) == 0
    rows_per_worker = total_rows // _NUM_WORKERS
    assert rows_per_worker % n_chunks == 0
    chunk = rows_per_worker // n_chunks
    mesh = plsc.VectorSubcoreMesh(
        core_axis_name="c", subcore_axis_name="s",
        num_cores=_NUM_CORES, num_subcores=_NUM_SUBCORES)

    @functools.partial(
        pl.kernel,
        mesh=mesh,
        out_type=jax.ShapeDtypeStruct((total_rows, d_model), jnp.float32),
        scratch_types=[
            pltpu.VMEM((rows_per_worker,), jnp.int32),
            pltpu.VMEM((rows_per_worker, d_model), jnp.float32),
            pltpu.SemaphoreType.DMA,
            pltpu.SemaphoreType.DMA,
        ],
    )
    def sc_gather(table_hbm, idx_hbm, out_hbm, idx_v, rows_v, gsem, wsem):
        wid = lax.axis_index("s") * _NUM_CORES + lax.axis_index("c")
        base = wid * rows_per_worker
        pltpu.sync_copy(idx_hbm.at[pl.ds(base, rows_per_worker)], idx_v)
        gathers, writebacks = [], []
        for c in range(n_chunks):
            g = pltpu.make_async_copy(
                table_hbm.at[idx_v.at[pl.ds(c * chunk, chunk)]],
                rows_v.at[pl.ds(c * chunk, chunk)], gsem)
            g.start()
            gathers.append(g)
        for c in range(n_chunks):
            gathers[c].wait()
            wb = pltpu.make_async_copy(
                rows_v.at[pl.ds(c * chunk, chunk)],
                out_hbm.at[pl.ds(base + c * chunk, chunk)], wsem)
            wb.start()
            writebacks.append(wb)
        for wb in writebacks:
            wb.wait()

    return sc_gather


def _matmul_body(lang_ref, x_ref, w_ref, b_ref, o_ref):
    del lang_ref
    acc = jax.lax.dot_general(
        x_ref[...], w_ref[0],
        dimension_numbers=(((1,), (1,)), ((), ())),
        preferred_element_type=jnp.float32)
    o_ref[...] = acc + b_ref[0]


@functools.lru_cache(maxsize=None)
def _make_tc_matmul(n_seq: int, seq_len: int, d_model: int, blk: int):
    n_tiles = seq_len // blk
    grid_spec = pltpu.PrefetchScalarGridSpec(
        num_scalar_prefetch=1,
        grid=(n_seq, n_tiles),
        in_specs=[
            pl.BlockSpec((blk, d_model),
                         lambda n, t, lang: (n * n_tiles + t, 0)),
            pl.BlockSpec((1, d_model, d_model),
                         lambda n, t, lang: (lang[n], 0, 0)),
            pl.BlockSpec((1, 1, d_model),
                         lambda n, t, lang: (lang[n], 0, 0)),
        ],
        out_specs=pl.BlockSpec((blk, d_model),
                               lambda n, t, lang: (n * n_tiles + t, 0)),
    )
    return pl.pallas_call(
        _matmul_body,
        grid_spec=grid_spec,
        out_shape=jax.ShapeDtypeStruct((n_seq * seq_len, d_model), jnp.float32),
    )


def kernel(sequences, embed_table, W, b):
    n_seq, seq_len = sequences.shape
    d_model = embed_table.shape[1]
    flat_idx = sequences.reshape(n_seq * seq_len).astype(jnp.int32)
    lang_ids = sequences[:, 0].astype(jnp.int32)
    rows = _make_sc_gather(n_seq * seq_len, d_model, 1)(embed_table, flat_idx)
    out = _make_tc_matmul(n_seq, seq_len, d_model, 2048)(
        lang_ids, rows, W, b.reshape(b.shape[0], 1, d_model))
    return out.reshape(n_seq, seq_len, d_model)
